# baseline (speedup 1.0000x reference)
import functools

import jax
import jax.numpy as jnp
from jax import lax
from jax.experimental import pallas as pl
from jax.experimental.pallas import tpu as pltpu
import jax.experimental.pallas.tpu_sc as plsc

N = 10000
E = 320000
D = 128

_NUM_CORES = 2
_NUM_SUBCORES = 16
_BLK = 80
_PER_TILE = E // _NUM_SUBCORES
_PER_WORKER = E // (_NUM_CORES * _NUM_SUBCORES)
_ZCHUNK = 200
_CTILES = 10
_CROWS = N // _CTILES


def _sc_mesh():
    return plsc.VectorSubcoreMesh(core_axis_name="c", subcore_axis_name="s")


def _scatter_body(ea_hbm, ei_hbm, z1d_hbm,
                  tab_src_hbm, tab_dst_hbm, cnt_src_hbm, cnt_dst_hbm,
                  idx_v, rows_v, ones_v, zcnt, table_sh, cnt_sh,
                  sem_i, sem_r, sem_s, sem_c):
    c = lax.axis_index("c")
    s = lax.axis_index("s")
    nblk = _PER_TILE // _BLK

    for k in range(_BLK // 16):
        ones_v[pl.ds(16 * k, 16)] = jnp.ones((16,), jnp.float32)
    for r in range(_BLK):
        for k in range(D // 16):
            rows_v[0, r, pl.ds(16 * k, 16)] = jnp.zeros((16,), jnp.float32)

    pltpu.sync_copy(z1d_hbm, zcnt)
    _tail = _CROWS % _BLK

    @pl.when(s < _CTILES)
    def _():
        base_n = s * _CROWS
        for k in range(_CROWS // _BLK):
            pltpu.sync_copy(rows_v.at[0],
                            table_sh.at[pl.ds(base_n + k * _BLK, _BLK)])
        pltpu.sync_copy(rows_v.at[0, pl.ds(0, _tail)],
                        table_sh.at[pl.ds(base_n + _CROWS - _tail, _tail)])
        pltpu.sync_copy(zcnt, cnt_sh.at[pl.ds(base_n, _CROWS)])

    plsc.subcore_barrier()

    base0 = s * _PER_TILE

    def _idx_src(t):
        return ei_hbm.at[pl.ds(c * E + base0 + t * _BLK, _BLK)]

    def _row_src(t):
        return ea_hbm.at[pl.ds(base0 + t * _BLK, _BLK)]

    def _issue_loads(t):
        b = t % 4
        pltpu.async_copy(_idx_src(t), idx_v.at[b], sem_i.at[b])
        pltpu.async_copy(_row_src(t), rows_v.at[b], sem_r.at[b])

    def _wait_scatters(b):
        pltpu.make_async_copy(
            rows_v.at[b], table_sh.at[idx_v.at[b]], sem_s.at[b]).wait()
        pltpu.make_async_copy(
            ones_v, cnt_sh.at[idx_v.at[b]], sem_c.at[b]).wait()

    _issue_loads(0)
    _issue_loads(1)

    @pl.loop(0, nblk)
    def _(j):
        b = j % 4
        pltpu.make_async_copy(_idx_src(j), idx_v.at[b], sem_i.at[b]).wait()
        pltpu.make_async_copy(_row_src(j), rows_v.at[b], sem_r.at[b]).wait()
        pltpu.async_copy(rows_v.at[b], table_sh.at[idx_v.at[b]], sem_s.at[b],
                         add=True)
        pltpu.async_copy(ones_v, cnt_sh.at[idx_v.at[b]], sem_c.at[b],
                         add=True)

        @pl.when(j + 2 < nblk)
        def _():
            @pl.when(j >= 2)
            def _():
                _wait_scatters((j + 2) % 4)

            _issue_loads(j + 2)

    for t in (nblk - 2, nblk - 1):
        _wait_scatters(t % 4)

    plsc.subcore_barrier()

    def _copy_out(tab_hbm, cnt_hbm):
        base_n = s * _CROWS
        for k in range(_CROWS // _BLK):
            r = base_n + k * _BLK
            pltpu.sync_copy(table_sh.at[pl.ds(r, _BLK)], rows_v.at[0])
            pltpu.sync_copy(rows_v.at[0], tab_hbm.at[pl.ds(r, _BLK)])
        r = base_n + _CROWS - _tail
        pltpu.sync_copy(table_sh.at[pl.ds(r, _tail)],
                        rows_v.at[0, pl.ds(0, _tail)])
        pltpu.sync_copy(rows_v.at[0, pl.ds(0, _tail)],
                        tab_hbm.at[pl.ds(r, _tail)])
        pltpu.sync_copy(cnt_sh.at[pl.ds(base_n, _CROWS)], zcnt)
        pltpu.sync_copy(zcnt, cnt_hbm.at[pl.ds(base_n, _CROWS)])

    @pl.when(jnp.logical_and(s < _CTILES, c == 0))
    def _():
        _copy_out(tab_src_hbm, cnt_src_hbm)

    @pl.when(jnp.logical_and(s < _CTILES, c == 1))
    def _():
        _copy_out(tab_dst_hbm, cnt_dst_hbm)


def _segment_sums(edge_attr, ei_flat):
    z1d = jnp.zeros((_CROWS,), jnp.float32)
    f = pl.kernel(
        _scatter_body,
        out_type=[
            jax.ShapeDtypeStruct((N, D), jnp.float32),
            jax.ShapeDtypeStruct((N, D), jnp.float32),
            jax.ShapeDtypeStruct((N,), jnp.float32),
            jax.ShapeDtypeStruct((N,), jnp.float32),
        ],
        mesh=_sc_mesh(),
        scratch_types=[
            pltpu.VMEM((4, _BLK), jnp.int32),
            pltpu.VMEM((4, _BLK, D), jnp.float32),
            pltpu.VMEM((_BLK,), jnp.float32),
            pltpu.VMEM((_CROWS,), jnp.float32),
            pltpu.VMEM_SHARED((N, D), jnp.float32),
            pltpu.VMEM_SHARED((N,), jnp.float32),
            pltpu.SemaphoreType.DMA((4,)),
            pltpu.SemaphoreType.DMA((4,)),
            pltpu.SemaphoreType.DMA((4,)),
            pltpu.SemaphoreType.DMA((4,)),
        ],
        name="seg_scatter_add",
    )
    return f(edge_attr, ei_flat, z1d)


def _gather_body(p_hbm, ei_hbm, out_hbm, idx_v, rows_v, sem_i, sem_g, sem_o):
    c = lax.axis_index("c")
    s = lax.axis_index("s")
    wid = s * _NUM_CORES + c
    base0 = wid * _PER_WORKER
    nblk = _PER_WORKER // _BLK

    def _idx_src(t):
        return ei_hbm.at[pl.ds(base0 + t * _BLK, _BLK)]

    def _out_dst(t):
        return out_hbm.at[pl.ds(base0 + t * _BLK, _BLK)]

    def _finish_block(t, b):
        pltpu.make_async_copy(
            p_hbm.at[idx_v.at[b]], rows_v.at[b], sem_g.at[b]).wait()
        pltpu.async_copy(rows_v.at[b], _out_dst(t), sem_o.at[b])

    _issue = lambda t: pltpu.async_copy(
        _idx_src(t), idx_v.at[t % 4], sem_i.at[t % 4])
    _issue(0)
    _issue(1)

    @pl.loop(0, nblk)
    def _(j):
        b = j % 4
        pltpu.make_async_copy(_idx_src(j), idx_v.at[b], sem_i.at[b]).wait()

        @pl.when(j >= 4)
        def _():
            pltpu.make_async_copy(
                rows_v.at[b], _out_dst(j - 4), sem_o.at[b]).wait()

        pltpu.async_copy(p_hbm.at[idx_v.at[b]], rows_v.at[b], sem_g.at[b])

        @pl.when(j >= 2)
        def _():
            _finish_block(j - 2, (j - 2) % 4)

        @pl.when(j + 2 < nblk)
        def _():
            _issue(j + 2)

    for t in (nblk - 2, nblk - 1):
        _finish_block(t, t % 4)
    for t in range(nblk - 4, nblk):
        b = t % 4
        pltpu.make_async_copy(rows_v.at[b], _out_dst(t), sem_o.at[b]).wait()


def _gather_rows(p, ei_flat):
    f = pl.kernel(
        _gather_body,
        out_type=jax.ShapeDtypeStruct((E, D), jnp.float32),
        mesh=_sc_mesh(),
        scratch_types=[
            pltpu.VMEM((4, _BLK), jnp.int32),
            pltpu.VMEM((4, _BLK, D), jnp.float32),
            pltpu.SemaphoreType.DMA((4,)),
            pltpu.SemaphoreType.DMA((4,)),
            pltpu.SemaphoreType.DMA((4,)),
        ],
        name="gather_p",
    )
    return f(p, ei_flat)


_NB = 1000


def _node_body(x_ref, ss_ref, sd_ref, cs_ref, cd_ref,
               wsrc_ref, wnv_ref, wev_ref, w4b_ref, bias_ref,
               node_ref, p_ref):
    xs = jnp.dot(x_ref[...], wsrc_ref[...], preferred_element_type=jnp.float32)
    la = sd_ref[...] / jnp.maximum(cd_ref[...], 1.0)
    a = ss_ref[...] + la
    v = (jnp.dot(a, wev_ref[...], preferred_element_type=jnp.float32)
         + (cs_ref[...] + 1.0)
         * jnp.dot(xs, wnv_ref[...], preferred_element_type=jnp.float32))
    nu = xs * v
    node_ref[...] = nu + bias_ref[...]
    p_ref[...] = jnp.dot(nu, w4b_ref[...], preferred_element_type=jnp.float32)


def _node_stage(x, ss, sd, cs, cd, w_src, w_nv, w_ev, w4b, bias):
    row = lambda i: (i, 0)
    full = lambda i: (0, 0)
    return pl.pallas_call(
        _node_body,
        grid=(N // _NB,),
        in_specs=[
            pl.BlockSpec((_NB, D), row),
            pl.BlockSpec((_NB, D), row),
            pl.BlockSpec((_NB, D), row),
            pl.BlockSpec((_NB, 1), row),
            pl.BlockSpec((_NB, 1), row),
            pl.BlockSpec((D, D), full),
            pl.BlockSpec((D, D), full),
            pl.BlockSpec((D, D), full),
            pl.BlockSpec((D, D), full),
            pl.BlockSpec((1, D), full),
        ],
        out_specs=[
            pl.BlockSpec((_NB, D), row),
            pl.BlockSpec((_NB, D), row),
        ],
        out_shape=[
            jax.ShapeDtypeStruct((N, D), jnp.float32),
            jax.ShapeDtypeStruct((N, D), jnp.float32),
        ],
        name="node_dense",
    )(x, ss, sd, cs, cd, w_src, w_nv, w_ev, w4b, bias)


_EB = 1600


def _ln(v, w, b):
    mu = jnp.mean(v, axis=-1, keepdims=True)
    var = jnp.mean((v - mu) ** 2, axis=-1, keepdims=True)
    return (v - mu) / jnp.sqrt(var) * w + b


def _edge_body(ea_ref, pg_ref, w4a_ref, w5_ref,
               l1w_ref, l1b_ref, l2w_ref, l2b_ref, ea2_ref, out_ref):
    bf = jnp.bfloat16
    ea = ea_ref[...]
    w5 = w5_ref[...].astype(bf)
    m = jnp.maximum(
        jnp.dot(ea.astype(bf), w4a_ref[...].astype(bf),
                preferred_element_type=jnp.float32)
        + pg_ref[...], 0.0)
    t = (jnp.dot(m.astype(bf), w5, preferred_element_type=jnp.float32)
         + ea2_ref[...])
    u = _ln(t, l1w_ref[...], l1b_ref[...])
    v = (jnp.dot(u.astype(bf), w5, preferred_element_type=jnp.float32)
         + ea2_ref[...])
    out_ref[...] = _ln(v, l2w_ref[...], l2b_ref[...])


def _edge_stage(edge_attr, pg, w4a, w5, l1w, l1b, l2w, l2b):
    row = lambda i: (i, 0)
    full = lambda i: (0, 0)
    return pl.pallas_call(
        _edge_body,
        grid=(E // _EB,),
        in_specs=[
            pl.BlockSpec((_EB, D), row),
            pl.BlockSpec((_EB, D), row),
            pl.BlockSpec((D, D), full),
            pl.BlockSpec((D, D), full),
            pl.BlockSpec((1, D), full),
            pl.BlockSpec((1, D), full),
            pl.BlockSpec((1, D), full),
            pl.BlockSpec((1, D), full),
            pl.BlockSpec((_EB, D), row),
        ],
        out_specs=pl.BlockSpec((_EB, D), row),
        out_shape=jax.ShapeDtypeStruct((E, D), jnp.float32),
        name="edge_pipeline",
    )(edge_attr, pg, w4a, w5, l1w, l1b, l2w, l2b, edge_attr)


def kernel(x, edge_index, edge_attr, W_src, W_nq, W_nv, W_nk, W_eq, W_ev,
           W_ek, W4, W5, ln1_w, ln1_b, ln2_w, ln2_b, bias):
    ei_flat = edge_index.astype(jnp.int32).reshape(-1)

    tab_src, tab_dst, cnt_src, cnt_dst = _segment_sums(edge_attr, ei_flat)

    node, p = _node_stage(
        x, tab_src, tab_dst,
        cnt_src.reshape(N, 1), cnt_dst.reshape(N, 1),
        W_src, W_nv, W_ev, W4[D:], bias.reshape(1, D))

    pg = _gather_rows(p, ei_flat)

    edge = _edge_stage(edge_attr, pg, W4[:D], W5,
                       ln1_w.reshape(1, D), ln1_b.reshape(1, D),
                       ln2_w.reshape(1, D), ln2_b.reshape(1, D))

    return node, edge


# speedup vs baseline: 1.1371x; 1.1371x over previous
import functools

import jax
import jax.numpy as jnp
from jax import lax
from jax.experimental import pallas as pl
from jax.experimental.pallas import tpu as pltpu
import jax.experimental.pallas.tpu_sc as plsc

N = 10000
E = 320000
D = 128

_NUM_CORES = 2
_NUM_SUBCORES = 16
_BLK = 80
_PER_TILE = E // _NUM_SUBCORES
_PER_WORKER = E // (_NUM_CORES * _NUM_SUBCORES)
_ZCHUNK = 200
_CTILES = 10
_CROWS = N // _CTILES


def _sc_mesh():
    return plsc.VectorSubcoreMesh(core_axis_name="c", subcore_axis_name="s")


def _scatter_body(ea_hbm, ei_hbm, z1d_hbm,
                  tab_src_hbm, tab_dst_hbm, cnt_src_hbm, cnt_dst_hbm,
                  idx_v, rows_v, ones_v, zcnt, table_sh, cnt_sh,
                  sem_i, sem_r, sem_s, sem_c):
    c = lax.axis_index("c")
    s = lax.axis_index("s")
    nblk = _PER_TILE // _BLK

    for k in range(_BLK // 16):
        ones_v[pl.ds(16 * k, 16)] = jnp.ones((16,), jnp.float32)
    for r in range(_BLK):
        for k in range(D // 16):
            rows_v[0, r, pl.ds(16 * k, 16)] = jnp.zeros((16,), jnp.float32)

    pltpu.sync_copy(z1d_hbm, zcnt)
    _tail = _CROWS % _BLK

    @pl.when(s < _CTILES)
    def _():
        base_n = s * _CROWS
        for k in range(_CROWS // _BLK):
            pltpu.sync_copy(rows_v.at[0],
                            table_sh.at[pl.ds(base_n + k * _BLK, _BLK)])
        pltpu.sync_copy(rows_v.at[0, pl.ds(0, _tail)],
                        table_sh.at[pl.ds(base_n + _CROWS - _tail, _tail)])
        pltpu.sync_copy(zcnt, cnt_sh.at[pl.ds(base_n, _CROWS)])

    plsc.subcore_barrier()

    base0 = s * _PER_TILE

    def _idx_src(t):
        return ei_hbm.at[pl.ds(c * E + base0 + t * _BLK, _BLK)]

    def _row_src(t):
        return ea_hbm.at[pl.ds(base0 + t * _BLK, _BLK)]

    def _issue_loads(t):
        b = t % 4
        pltpu.async_copy(_idx_src(t), idx_v.at[b], sem_i.at[b])
        pltpu.async_copy(_row_src(t), rows_v.at[b], sem_r.at[b])

    def _wait_scatters(b):
        pltpu.make_async_copy(
            rows_v.at[b], table_sh.at[idx_v.at[b]], sem_s.at[b]).wait()
        pltpu.make_async_copy(
            ones_v, cnt_sh.at[idx_v.at[b]], sem_c.at[b]).wait()

    _issue_loads(0)
    _issue_loads(1)

    @pl.loop(0, nblk)
    def _(j):
        b = j % 4
        pltpu.make_async_copy(_idx_src(j), idx_v.at[b], sem_i.at[b]).wait()
        pltpu.make_async_copy(_row_src(j), rows_v.at[b], sem_r.at[b]).wait()
        pltpu.async_copy(rows_v.at[b], table_sh.at[idx_v.at[b]], sem_s.at[b],
                         add=True)
        pltpu.async_copy(ones_v, cnt_sh.at[idx_v.at[b]], sem_c.at[b],
                         add=True)

        @pl.when(j + 2 < nblk)
        def _():
            @pl.when(j >= 2)
            def _():
                _wait_scatters((j + 2) % 4)

            _issue_loads(j + 2)

    for t in (nblk - 2, nblk - 1):
        _wait_scatters(t % 4)

    plsc.subcore_barrier()

    def _copy_out(tab_hbm, cnt_hbm):
        base_n = s * _CROWS
        for k in range(_CROWS // _BLK):
            r = base_n + k * _BLK
            pltpu.sync_copy(table_sh.at[pl.ds(r, _BLK)], rows_v.at[0])
            pltpu.sync_copy(rows_v.at[0], tab_hbm.at[pl.ds(r, _BLK)])
        r = base_n + _CROWS - _tail
        pltpu.sync_copy(table_sh.at[pl.ds(r, _tail)],
                        rows_v.at[0, pl.ds(0, _tail)])
        pltpu.sync_copy(rows_v.at[0, pl.ds(0, _tail)],
                        tab_hbm.at[pl.ds(r, _tail)])
        pltpu.sync_copy(cnt_sh.at[pl.ds(base_n, _CROWS)], zcnt)
        pltpu.sync_copy(zcnt, cnt_hbm.at[pl.ds(base_n, _CROWS)])

    @pl.when(jnp.logical_and(s < _CTILES, c == 0))
    def _():
        _copy_out(tab_src_hbm, cnt_src_hbm)

    @pl.when(jnp.logical_and(s < _CTILES, c == 1))
    def _():
        _copy_out(tab_dst_hbm, cnt_dst_hbm)


def _segment_sums(edge_attr, ei_flat):
    z1d = jnp.zeros((_CROWS,), jnp.float32)
    f = pl.kernel(
        _scatter_body,
        out_type=[
            jax.ShapeDtypeStruct((N, D), jnp.float32),
            jax.ShapeDtypeStruct((N, D), jnp.float32),
            jax.ShapeDtypeStruct((N,), jnp.float32),
            jax.ShapeDtypeStruct((N,), jnp.float32),
        ],
        mesh=_sc_mesh(),
        scratch_types=[
            pltpu.VMEM((4, _BLK), jnp.int32),
            pltpu.VMEM((4, _BLK, D), jnp.float32),
            pltpu.VMEM((_BLK,), jnp.float32),
            pltpu.VMEM((_CROWS,), jnp.float32),
            pltpu.VMEM_SHARED((N, D), jnp.float32),
            pltpu.VMEM_SHARED((N,), jnp.float32),
            pltpu.SemaphoreType.DMA((4,)),
            pltpu.SemaphoreType.DMA((4,)),
            pltpu.SemaphoreType.DMA((4,)),
            pltpu.SemaphoreType.DMA((4,)),
        ],
        name="seg_scatter_add",
    )
    return f(edge_attr, ei_flat, z1d)


def _gather_body(p_hbm, ei_hbm, out_hbm, idx_v, rows_v, sem_i, sem_g, sem_o):
    c = lax.axis_index("c")
    s = lax.axis_index("s")
    wid = s * _NUM_CORES + c
    base0 = wid * _PER_WORKER
    nblk = _PER_WORKER // _BLK

    def _idx_src(t):
        return ei_hbm.at[pl.ds(base0 + t * _BLK, _BLK)]

    def _out_dst(t):
        return out_hbm.at[pl.ds(base0 + t * _BLK, _BLK)]

    def _finish_block(t, b):
        pltpu.make_async_copy(
            p_hbm.at[idx_v.at[b]], rows_v.at[b], sem_g.at[b]).wait()
        pltpu.async_copy(rows_v.at[b], _out_dst(t), sem_o.at[b])

    _issue = lambda t: pltpu.async_copy(
        _idx_src(t), idx_v.at[t % 4], sem_i.at[t % 4])
    _issue(0)
    _issue(1)

    @pl.loop(0, nblk)
    def _(j):
        b = j % 4
        pltpu.make_async_copy(_idx_src(j), idx_v.at[b], sem_i.at[b]).wait()

        @pl.when(j >= 4)
        def _():
            pltpu.make_async_copy(
                rows_v.at[b], _out_dst(j - 4), sem_o.at[b]).wait()

        pltpu.async_copy(p_hbm.at[idx_v.at[b]], rows_v.at[b], sem_g.at[b])

        @pl.when(j >= 2)
        def _():
            _finish_block(j - 2, (j - 2) % 4)

        @pl.when(j + 2 < nblk)
        def _():
            _issue(j + 2)

    for t in (nblk - 2, nblk - 1):
        _finish_block(t, t % 4)
    for t in range(nblk - 4, nblk):
        b = t % 4
        pltpu.make_async_copy(rows_v.at[b], _out_dst(t), sem_o.at[b]).wait()


def _gather_rows(p, ei_flat):
    f = pl.kernel(
        _gather_body,
        out_type=jax.ShapeDtypeStruct((E, D), jnp.float32),
        mesh=_sc_mesh(),
        scratch_types=[
            pltpu.VMEM((4, _BLK), jnp.int32),
            pltpu.VMEM((4, _BLK, D), jnp.float32),
            pltpu.SemaphoreType.DMA((4,)),
            pltpu.SemaphoreType.DMA((4,)),
            pltpu.SemaphoreType.DMA((4,)),
        ],
        name="gather_p",
    )
    return f(p, ei_flat)


_NB = 1000


def _node_body(x_ref, ss_ref, sd_ref, cs_ref, cd_ref,
               wsrc_ref, wnv_ref, wev_ref, w4b_ref, bias_ref,
               node_ref, p_ref):
    xs = jnp.dot(x_ref[...], wsrc_ref[...], preferred_element_type=jnp.float32)
    la = sd_ref[...] / jnp.maximum(cd_ref[...], 1.0)
    a = ss_ref[...] + la
    v = (jnp.dot(a, wev_ref[...], preferred_element_type=jnp.float32)
         + (cs_ref[...] + 1.0)
         * jnp.dot(xs, wnv_ref[...], preferred_element_type=jnp.float32))
    nu = xs * v
    node_ref[...] = nu + bias_ref[...]
    p_ref[...] = jnp.dot(nu, w4b_ref[...], preferred_element_type=jnp.float32)


def _node_stage(x, ss, sd, cs, cd, w_src, w_nv, w_ev, w4b, bias):
    row = lambda i: (i, 0)
    full = lambda i: (0, 0)
    return pl.pallas_call(
        _node_body,
        grid=(N // _NB,),
        in_specs=[
            pl.BlockSpec((_NB, D), row),
            pl.BlockSpec((_NB, D), row),
            pl.BlockSpec((_NB, D), row),
            pl.BlockSpec((_NB, 1), row),
            pl.BlockSpec((_NB, 1), row),
            pl.BlockSpec((D, D), full),
            pl.BlockSpec((D, D), full),
            pl.BlockSpec((D, D), full),
            pl.BlockSpec((D, D), full),
            pl.BlockSpec((1, D), full),
        ],
        out_specs=[
            pl.BlockSpec((_NB, D), row),
            pl.BlockSpec((_NB, D), row),
        ],
        out_shape=[
            jax.ShapeDtypeStruct((N, D), jnp.float32),
            jax.ShapeDtypeStruct((N, D), jnp.float32),
        ],
        name="node_dense",
    )(x, ss, sd, cs, cd, w_src, w_nv, w_ev, w4b, bias)


_EB = 6400


def _ln(v, w, b):
    mu = jnp.mean(v, axis=-1, keepdims=True)
    var = jnp.mean((v - mu) ** 2, axis=-1, keepdims=True)
    return (v - mu) / jnp.sqrt(var) * w + b


def _edge_body(ea_ref, pg_ref, w4a_ref, w5_ref,
               l1w_ref, l1b_ref, l2w_ref, l2b_ref, ea2_ref, out_ref):
    bf = jnp.bfloat16
    ea = ea_ref[...]
    w5 = w5_ref[...].astype(bf)
    m = jnp.maximum(
        jnp.dot(ea.astype(bf), w4a_ref[...].astype(bf),
                preferred_element_type=jnp.float32)
        + pg_ref[...], 0.0)
    t = (jnp.dot(m.astype(bf), w5, preferred_element_type=jnp.float32)
         + ea2_ref[...])
    u = _ln(t, l1w_ref[...], l1b_ref[...])
    v = (jnp.dot(u.astype(bf), w5, preferred_element_type=jnp.float32)
         + ea2_ref[...])
    out_ref[...] = _ln(v, l2w_ref[...], l2b_ref[...])


def _edge_stage(edge_attr, pg, w4a, w5, l1w, l1b, l2w, l2b):
    row = lambda i: (i, 0)
    full = lambda i: (0, 0)
    return pl.pallas_call(
        _edge_body,
        grid=(E // _EB,),
        in_specs=[
            pl.BlockSpec((_EB, D), row),
            pl.BlockSpec((_EB, D), row),
            pl.BlockSpec((D, D), full),
            pl.BlockSpec((D, D), full),
            pl.BlockSpec((1, D), full),
            pl.BlockSpec((1, D), full),
            pl.BlockSpec((1, D), full),
            pl.BlockSpec((1, D), full),
            pl.BlockSpec((_EB, D), row),
        ],
        out_specs=pl.BlockSpec((_EB, D), row),
        out_shape=jax.ShapeDtypeStruct((E, D), jnp.float32),
        name="edge_pipeline",
    )(edge_attr, pg, w4a, w5, l1w, l1b, l2w, l2b, edge_attr)


def kernel(x, edge_index, edge_attr, W_src, W_nq, W_nv, W_nk, W_eq, W_ev,
           W_ek, W4, W5, ln1_w, ln1_b, ln2_w, ln2_b, bias):
    ei_flat = edge_index.astype(jnp.int32).reshape(-1)

    tab_src, tab_dst, cnt_src, cnt_dst = _segment_sums(edge_attr, ei_flat)

    node, p = _node_stage(
        x, tab_src, tab_dst,
        cnt_src.reshape(N, 1), cnt_dst.reshape(N, 1),
        W_src, W_nv, W_ev, W4[D:], bias.reshape(1, D))

    pg = _gather_rows(p, ei_flat)

    edge = _edge_stage(edge_attr, pg, W4[:D], W5,
                       ln1_w.reshape(1, D), ln1_b.reshape(1, D),
                       ln2_w.reshape(1, D), ln2_b.reshape(1, D))

    return node, edge
